# baseline (speedup 1.0000x reference)
import functools

import jax
import jax.numpy as jnp
from jax import lax
from jax.experimental import pallas as pl
from jax.experimental.pallas import tpu as pltpu
from jax.experimental.pallas import tpu_sc as plsc

B, L, V, D, P, NB = 1024, 50, 100000, 512, 32, 51
N = B * L
NBP = 64

K = 5
LCH = L // K
NTOK = N // K

NC, NS = 2, 16
NW = NC * NS
ROWS_PER_W = NTOK // NW
CHUNK = 80
NCHUNK = ROWS_PER_W // CHUNK


def _sc_gather(table, ids):
  mesh = plsc.VectorSubcoreMesh(core_axis_name="c", subcore_axis_name="s")

  @functools.partial(
      pl.kernel,
      mesh=mesh,
      out_type=jax.ShapeDtypeStruct((NTOK, D), jnp.float32),
      scratch_types=[
          pltpu.VMEM((ROWS_PER_W,), jnp.int32),
          pltpu.VMEM((CHUNK, D), jnp.float32),
          pltpu.VMEM((CHUNK, D), jnp.float32),
          pltpu.SemaphoreType.DMA,
          pltpu.SemaphoreType.DMA,
          pltpu.SemaphoreType.DMA,
      ],
  )
  def k(table_hbm, idx_hbm, out_hbm, idx_v, buf0, buf1, gsem, osem0, osem1):
    wid = lax.axis_index("s") * NC + lax.axis_index("c")
    base = wid * ROWS_PER_W
    pltpu.sync_copy(idx_hbm.at[pl.ds(base, ROWS_PER_W)], idx_v)

    bufs = (buf0, buf1)
    osems = (osem0, osem1)

    @pl.loop(0, NCHUNK, step=2)
    def _(c):
      for b in range(2):
        i = c + b
        buf, osem = bufs[b], osems[b]

        @pl.when(i >= 2)
        def _():
          pltpu.make_async_copy(
              buf, out_hbm.at[pl.ds(base + (i - 2) * CHUNK, CHUNK)], osem
          ).wait()

        idx_slice = idx_v.at[pl.ds(i * CHUNK, CHUNK)]
        pltpu.async_copy(table_hbm.at[idx_slice], buf, gsem).wait()
        pltpu.async_copy(buf, out_hbm.at[pl.ds(base + i * CHUNK, CHUNK)], osem)

    for b in range(2):
      i = NCHUNK - 2 + b
      pltpu.make_async_copy(
          bufs[b], out_hbm.at[pl.ds(base + i * CHUNK, CHUNK)], osems[b]
      ).wait()

  return k(table, ids)


BB = 512


def _tc_body(g_ref, vid_ref, pos_ref, bin_ref, wpos_ref, gam_ref, bet_ref,
             o_ref):
  x = g_ref[...]
  vid = vid_ref[0]
  onehot_t = (vid == lax.broadcasted_iota(jnp.int32, (NBP, BB), 0)
              ).astype(jnp.float32)
  cdims = (((0,), (0,)), ((), ()))
  x = x + lax.dot_general(onehot_t, bin_ref[...], cdims,
                          preferred_element_type=jnp.float32)
  x = x + lax.dot_general(pos_ref[0], wpos_ref[...], cdims,
                          preferred_element_type=jnp.float32)
  mu = jnp.mean(x, axis=1, keepdims=True)
  xc = x - mu
  var = jnp.mean(xc * xc, axis=1, keepdims=True)
  inv = lax.rsqrt(var + 1e-5)
  o_ref[0] = xc * inv * gam_ref[...] + bet_ref[...]


def _tc_body_carry(carry_ref, *refs):
  del carry_ref
  _tc_body(*refs)


def _tc_compose(c, carry, gathered, vids, post, bin_pad, w_pos, gamma2,
                beta2):
  specs = [
      pl.BlockSpec((BB, D), lambda i, j: (i * (B // BB) + j, 0)),
      pl.BlockSpec((1, 1, BB), lambda i, j, c=c: (c * LCH + i, 0, j)),
      pl.BlockSpec((1, P, BB), lambda i, j, c=c: (c * LCH + i, 0, j)),
      pl.BlockSpec((NBP, D), lambda i, j: (0, 0)),
      pl.BlockSpec((P, D), lambda i, j: (0, 0)),
      pl.BlockSpec((1, D), lambda i, j: (0, 0)),
      pl.BlockSpec((1, D), lambda i, j: (0, 0)),
  ]
  out_spec = pl.BlockSpec((1, BB, D), lambda i, j, c=c: (c * LCH + i, j, 0))
  out_shape = jax.ShapeDtypeStruct((L, B, D), jnp.float32)
  args = (gathered, vids, post, bin_pad, w_pos, gamma2, beta2)
  if carry is None:
    return pl.pallas_call(
        _tc_body, grid=(LCH, B // BB), in_specs=specs, out_specs=out_spec,
        out_shape=out_shape,
    )(*args)
  return pl.pallas_call(
      _tc_body_carry, grid=(LCH, B // BB),
      in_specs=[pl.BlockSpec(memory_space=pl.ANY)] + specs,
      out_specs=out_spec, out_shape=out_shape,
      input_output_aliases={0: 0},
  )(carry, *args)


@jax.jit
def kernel(gene_ids, pos_emb, value_ids, gene_table, bin_table, W_pos, gamma,
           beta):
  ids = gene_ids.T.reshape(-1).astype(jnp.int32)
  vids = value_ids.T.reshape(L, 1, B).astype(jnp.int32)
  post = pos_emb.transpose(1, 2, 0)
  bin_pad = jnp.pad(bin_table, ((0, NBP - NB), (0, 0)))
  gamma2 = gamma.reshape(1, D)
  beta2 = beta.reshape(1, D)

  gathered = [_sc_gather(gene_table, ids[c * NTOK:(c + 1) * NTOK])
              for c in range(K)]
  out3 = None
  for c in range(K):
    out3 = _tc_compose(c, out3, gathered[c], vids, post, bin_pad, W_pos,
                       gamma2, beta2)
  return out3.transpose(1, 0, 2)


# speedup vs baseline: 1.1353x; 1.1353x over previous
import functools

import jax
import jax.numpy as jnp
from jax import lax
from jax.experimental import pallas as pl
from jax.experimental.pallas import tpu as pltpu
from jax.experimental.pallas import tpu_sc as plsc

B, L, V, D, P, NB = 1024, 50, 100000, 512, 32, 51
N = B * L
NBP = 64

K = 5
LCH = L // K
NTOK = N // K

NC, NS = 2, 16
NW = NC * NS
ROWS_PER_W = NTOK // NW
CHUNK = 80
NCHUNK = ROWS_PER_W // CHUNK


def _sc_gather(table, ids):
  mesh = plsc.VectorSubcoreMesh(core_axis_name="c", subcore_axis_name="s")

  @functools.partial(
      pl.kernel,
      mesh=mesh,
      out_type=jax.ShapeDtypeStruct((NTOK, D), jnp.float32),
      scratch_types=[
          pltpu.VMEM((ROWS_PER_W,), jnp.int32),
          pltpu.VMEM((CHUNK, D), jnp.float32),
          pltpu.VMEM((CHUNK, D), jnp.float32),
          pltpu.SemaphoreType.DMA,
          pltpu.SemaphoreType.DMA,
          pltpu.SemaphoreType.DMA,
      ],
  )
  def k(table_hbm, idx_hbm, out_hbm, idx_v, buf0, buf1, gsem, osem0, osem1):
    wid = lax.axis_index("s") * NC + lax.axis_index("c")
    base = wid * ROWS_PER_W
    pltpu.sync_copy(idx_hbm.at[pl.ds(base, ROWS_PER_W)], idx_v)

    bufs = (buf0, buf1)
    osems = (osem0, osem1)

    @pl.loop(0, NCHUNK, step=2)
    def _(c):
      for b in range(2):
        i = c + b
        buf, osem = bufs[b], osems[b]

        @pl.when(i >= 2)
        def _():
          pltpu.make_async_copy(
              buf, out_hbm.at[pl.ds(base + (i - 2) * CHUNK, CHUNK)], osem
          ).wait()

        idx_slice = idx_v.at[pl.ds(i * CHUNK, CHUNK)]
        pltpu.async_copy(table_hbm.at[idx_slice], buf, gsem).wait()
        pltpu.async_copy(buf, out_hbm.at[pl.ds(base + i * CHUNK, CHUNK)], osem)

    for b in range(2):
      i = NCHUNK - 2 + b
      pltpu.make_async_copy(
          bufs[b], out_hbm.at[pl.ds(base + i * CHUNK, CHUNK)], osems[b]
      ).wait()

  return k(table, ids)


def _tc_body(g_ref, vid_ref, pos_ref, bin_ref, wpos_ref, gam_ref, bet_ref,
             o_ref):
  x = g_ref[...]
  vid = vid_ref[0]
  onehot_t = (vid == lax.broadcasted_iota(jnp.int32, (NBP, B), 0)
              ).astype(jnp.float32)
  cdims = (((0,), (0,)), ((), ()))
  x = x + lax.dot_general(onehot_t, bin_ref[...], cdims,
                          preferred_element_type=jnp.float32)
  x = x + lax.dot_general(pos_ref[0], wpos_ref[...], cdims,
                          preferred_element_type=jnp.float32)
  mu = jnp.mean(x, axis=1, keepdims=True)
  xc = x - mu
  var = jnp.mean(xc * xc, axis=1, keepdims=True)
  inv = lax.rsqrt(var + 1e-5)
  o_ref[0] = xc * inv * gam_ref[...] + bet_ref[...]


def _tc_body_carry(carry_ref, *refs):
  del carry_ref
  _tc_body(*refs)


def _tc_compose(c, carry, gathered, vids, post, bin_pad, w_pos, gamma2,
                beta2):
  specs = [
      pl.BlockSpec((B, D), lambda i: (i, 0)),
      pl.BlockSpec((1, 1, B), lambda i, c=c: (c * LCH + i, 0, 0)),
      pl.BlockSpec((1, P, B), lambda i, c=c: (c * LCH + i, 0, 0)),
      pl.BlockSpec((NBP, D), lambda i: (0, 0)),
      pl.BlockSpec((P, D), lambda i: (0, 0)),
      pl.BlockSpec((1, D), lambda i: (0, 0)),
      pl.BlockSpec((1, D), lambda i: (0, 0)),
  ]
  out_spec = pl.BlockSpec((1, B, D), lambda i, c=c: (c * LCH + i, 0, 0))
  out_shape = jax.ShapeDtypeStruct((L, B, D), jnp.float32)
  args = (gathered, vids, post, bin_pad, w_pos, gamma2, beta2)
  if carry is None:
    return pl.pallas_call(
        _tc_body, grid=(LCH,), in_specs=specs, out_specs=out_spec,
        out_shape=out_shape,
    )(*args)
  return pl.pallas_call(
      _tc_body_carry, grid=(LCH,),
      in_specs=[pl.BlockSpec(memory_space=pl.ANY)] + specs,
      out_specs=out_spec, out_shape=out_shape,
      input_output_aliases={0: 0},
  )(carry, *args)


@jax.jit
def kernel(gene_ids, pos_emb, value_ids, gene_table, bin_table, W_pos, gamma,
           beta):
  ids = gene_ids.T.reshape(-1).astype(jnp.int32)
  vids = value_ids.T.reshape(L, 1, B).astype(jnp.int32)
  post = pos_emb.transpose(1, 2, 0)
  bin_pad = jnp.pad(bin_table, ((0, NBP - NB), (0, 0)))
  gamma2 = gamma.reshape(1, D)
  beta2 = beta.reshape(1, D)

  gathered = [_sc_gather(gene_table, ids[c * NTOK:(c + 1) * NTOK])
              for c in range(K)]
  out3 = None
  for c in range(K):
    out3 = _tc_compose(c, out3, gathered[c], vids, post, bin_pad, W_pos,
                       gamma2, beta2)
  return out3.transpose(1, 0, 2)
